# baseline (speedup 1.0000x reference)
import functools

import jax
import jax.numpy as jnp
from jax.experimental import pallas as pl
from jax.experimental.pallas import tpu as pltpu

TOKENS = 2048
F = 768
E = 16
K = 4
H = 3072
SH = 6144
BT = 128
P = 10240
NT = P // BT
TT = 256
NTT = TOKENS // TT
PC = 2048
NPC = P // PC


def _front_kernel(x_ref, w0_ref, b0_ref, wg_ref, bg_ref, wu_ref, bu_ref,
                  wgate_ref, h_ref, tw_ref, ti_ref):
    f32 = jnp.float32
    x = x_ref[...]
    h0 = jax.lax.dot_general(x, w0_ref[...], (((1,), (1,)), ((), ())),
                             preferred_element_type=f32) + b0_ref[...][None, :]
    g = jax.lax.dot_general(h0, wg_ref[...], (((1,), (1,)), ((), ())),
                            preferred_element_type=f32) + bg_ref[...][None, :]
    u = jax.lax.dot_general(h0, wu_ref[...], (((1,), (1,)), ((), ())),
                            preferred_element_type=f32) + bu_ref[...][None, :]
    h = (g * jax.nn.sigmoid(g)) * u
    h_ref[...] = h
    logits = jax.lax.dot_general(h, wgate_ref[...], (((1,), (1,)), ((), ())),
                                 preferred_element_type=f32)
    m = jnp.max(logits, axis=-1, keepdims=True)
    p = jnp.exp(logits - m)
    p = p / jnp.sum(p, axis=-1, keepdims=True)
    cols = jax.lax.broadcasted_iota(jnp.int32, p.shape, 1)
    tws, tis = [], []
    for _ in range(K):
        mx = jnp.max(p, axis=-1, keepdims=True)
        idx = jnp.min(jnp.where(p == mx, cols, E), axis=-1, keepdims=True)
        tws.append(mx)
        tis.append(idx)
        p = jnp.where(cols == idx, -1.0, p)
    tw_ref[...] = jnp.concatenate(tws, axis=-1)
    ti_ref[...] = jnp.concatenate(tis, axis=-1)


def _route_kernel(ti_ref, pos_ref, te_ref):
    f32, i32, bf16 = jnp.float32, jnp.int32, jnp.bfloat16
    ti = ti_ref[...]
    iota_e = jax.lax.broadcasted_iota(i32, (TOKENS, E), 1)
    ohs = [(ti[:, k:k + 1] == iota_e).astype(f32) for k in range(K)]
    rowsum = ohs[0] + ohs[1] + ohs[2] + ohs[3]
    ri = jax.lax.broadcasted_iota(i32, (TOKENS, TOKENS), 0)
    ci = jax.lax.broadcasted_iota(i32, (TOKENS, TOKENS), 1)
    tril = (ci < ri).astype(bf16)
    prefix = jnp.dot(tril, rowsum.astype(bf16), preferred_element_type=f32)
    counts = jnp.sum(rowsum, axis=0, keepdims=True)
    c_pad = ((counts.astype(i32) + BT - 1) // BT) * BT
    ue = jax.lax.broadcasted_iota(i32, (E, E), 0)
    ve = jax.lax.broadcasted_iota(i32, (E, E), 1)
    upper = (ue < ve).astype(f32)
    pstart = jnp.dot(c_pad.astype(f32), upper,
                     preferred_element_type=f32)
    poss = []
    for k in range(K):
        r_k = jnp.sum(ohs[k] * prefix, axis=1, keepdims=True)
        p_k = jnp.sum(ohs[k] * pstart, axis=1, keepdims=True)
        poss.append(r_k + p_k)
    pos_ref[...] = jnp.concatenate(poss, axis=1).astype(i32)
    base = (jax.lax.broadcasted_iota(i32, (NT, 1), 0) * BT).astype(f32)
    cmp = (pstart <= base).astype(i32)
    te_ref[...] = (jnp.sum(cmp, axis=1, keepdims=True) - 1).astype(i32)


def _expert_kernel(te_ref, pos_ref, h_ref, w1_ref, w3_ref, w2_ref, rows_ref):
    del te_ref
    f32, i32, bf16 = jnp.float32, jnp.int32, jnp.bfloat16
    i = pl.program_id(0)
    sidr = jax.lax.broadcasted_iota(i32, (TOKENS, BT), 1) + i * BT
    dbool = (pos_ref[:, 0:1] == sidr)
    for k in range(1, K):
        dbool = dbool | (pos_ref[:, k:k + 1] == sidr)
    onehot_t = dbool.astype(bf16)
    hs = jax.lax.dot_general(onehot_t, h_ref[...], (((0,), (0,)), ((), ())),
                             preferred_element_type=f32)
    g = jax.lax.dot_general(hs, w1_ref[0], (((1,), (1,)), ((), ())),
                            preferred_element_type=f32)
    u = jax.lax.dot_general(hs, w3_ref[0], (((1,), (1,)), ((), ())),
                            preferred_element_type=f32)
    a = ((g * jax.nn.sigmoid(g)) * u).astype(bf16)
    rows_ref[...] = jax.lax.dot_general(
        a, w2_ref[0], (((1,), (1,)), ((), ())),
        preferred_element_type=f32).astype(bf16)


def _combine_kernel(pos_ref, tw_ref, rows_ref, y_ref):
    j = pl.program_id(1)

    @pl.when(j == 0)
    def _():
        y_ref[...] = jnp.zeros_like(y_ref)

    base = j * PC
    iota = jax.lax.broadcasted_iota(jnp.int32, (TT, PC), 1) + base
    m = jnp.zeros((TT, PC), jnp.float32)
    for k in range(K):
        pk = pos_ref[:, k][:, None]
        wk = tw_ref[:, k][:, None]
        m = m + jnp.where(pk == iota, wk, 0.0)
    y_ref[...] += jnp.dot(m.astype(jnp.bfloat16), rows_ref[...],
                          preferred_element_type=jnp.float32)


def _shared_kernel(h_ref, w1_ref, w3_ref, w2_ref, y_ref, out_ref):
    f32 = jnp.float32
    h = h_ref[...]
    g = jax.lax.dot_general(h, w1_ref[...], (((1,), (1,)), ((), ())),
                            preferred_element_type=f32)
    u = jax.lax.dot_general(h, w3_ref[...], (((1,), (1,)), ((), ())),
                            preferred_element_type=f32)
    a = ((g * jax.nn.sigmoid(g)) * u).astype(jnp.bfloat16)
    out_ref[...] = y_ref[...] + jax.lax.dot_general(
        a, w2_ref[...], (((1,), (1,)), ((), ())),
        preferred_element_type=f32)


def kernel(x, W0, b0, Wg, bg, Wu, bu, Wgate, we1, we3, we2, ws1, ws3, ws2):
    f32, bf16, i32 = jnp.float32, jnp.bfloat16, jnp.int32

    h, tw, ti = pl.pallas_call(
        _front_kernel,
        grid=(NTT,),
        in_specs=[
            pl.BlockSpec((TT, F), lambda i: (i, 0)),
            pl.BlockSpec((F, F), lambda i: (0, 0)),
            pl.BlockSpec((F,), lambda i: (0,)),
            pl.BlockSpec((F, F), lambda i: (0, 0)),
            pl.BlockSpec((F,), lambda i: (0,)),
            pl.BlockSpec((F, F), lambda i: (0, 0)),
            pl.BlockSpec((F,), lambda i: (0,)),
            pl.BlockSpec((E, F), lambda i: (0, 0)),
        ],
        out_specs=[
            pl.BlockSpec((TT, F), lambda i: (i, 0)),
            pl.BlockSpec((TT, K), lambda i: (i, 0)),
            pl.BlockSpec((TT, K), lambda i: (i, 0)),
        ],
        out_shape=[
            jax.ShapeDtypeStruct((TOKENS, F), f32),
            jax.ShapeDtypeStruct((TOKENS, K), f32),
            jax.ShapeDtypeStruct((TOKENS, K), i32),
        ],
    )(x, W0, b0, Wg, bg, Wu, bu, Wgate)

    pos, te = pl.pallas_call(
        _route_kernel,
        grid=(1,),
        in_specs=[pl.BlockSpec((TOKENS, K), lambda i: (0, 0))],
        out_specs=[
            pl.BlockSpec((TOKENS, K), lambda i: (0, 0)),
            pl.BlockSpec((NT, 1), lambda i: (0, 0)),
        ],
        out_shape=[
            jax.ShapeDtypeStruct((TOKENS, K), i32),
            jax.ShapeDtypeStruct((NT, 1), i32),
        ],
    )(ti)

    h_b = h.astype(bf16)
    we2_b = we2.astype(bf16)
    ws1_b, ws3_b, ws2_b = ws1.astype(bf16), ws3.astype(bf16), ws2.astype(bf16)

    rows = pl.pallas_call(
        _expert_kernel,
        grid_spec=pltpu.PrefetchScalarGridSpec(
            num_scalar_prefetch=1,
            grid=(NT,),
            in_specs=[
                pl.BlockSpec((TOKENS, K), lambda i, te: (0, 0)),
                pl.BlockSpec((TOKENS, F), lambda i, te: (0, 0)),
                pl.BlockSpec((1, H, F), lambda i, te: (te[i, 0], 0, 0)),
                pl.BlockSpec((1, H, F), lambda i, te: (te[i, 0], 0, 0)),
                pl.BlockSpec((1, F, H), lambda i, te: (te[i, 0], 0, 0)),
            ],
            out_specs=pl.BlockSpec((BT, F), lambda i, te: (i, 0)),
        ),
        out_shape=jax.ShapeDtypeStruct((P, F), bf16),
    )(te, pos, h_b, we1, we3, we2_b)

    y = pl.pallas_call(
        _combine_kernel,
        grid=(NTT, NPC),
        in_specs=[
            pl.BlockSpec((TT, K), lambda i, j: (i, 0)),
            pl.BlockSpec((TT, K), lambda i, j: (i, 0)),
            pl.BlockSpec((PC, F), lambda i, j: (j, 0)),
        ],
        out_specs=pl.BlockSpec((TT, F), lambda i, j: (i, 0)),
        out_shape=jax.ShapeDtypeStruct((TOKENS, F), f32),
    )(pos, tw, rows)

    out = pl.pallas_call(
        _shared_kernel,
        grid=(NTT,),
        in_specs=[
            pl.BlockSpec((TT, F), lambda i: (i, 0)),
            pl.BlockSpec((SH, F), lambda i: (0, 0)),
            pl.BlockSpec((SH, F), lambda i: (0, 0)),
            pl.BlockSpec((F, SH), lambda i: (0, 0)),
            pl.BlockSpec((TT, F), lambda i: (i, 0)),
        ],
        out_specs=pl.BlockSpec((TT, F), lambda i: (i, 0)),
        out_shape=jax.ShapeDtypeStruct((TOKENS, F), f32),
    )(h_b, ws1_b, ws3_b, ws2_b, y)

    return out


# speedup vs baseline: 1.1301x; 1.1301x over previous
import functools

import jax
import jax.numpy as jnp
from jax.experimental import pallas as pl
from jax.experimental.pallas import tpu as pltpu

TOKENS = 2048
F = 768
E = 16
K = 4
H = 3072
SH = 6144
BT = 128
P = 10240
NT = P // BT
TT = 256
NTT = TOKENS // TT
PC = 2048
NPC = P // PC


def _front_kernel(x_ref, w0_ref, b0_ref, wg_ref, bg_ref, wu_ref, bu_ref,
                  wgate_ref, h_ref, tw_ref, ti_ref):
    f32 = jnp.float32
    x = x_ref[...]
    h0 = jax.lax.dot_general(x, w0_ref[...], (((1,), (1,)), ((), ())),
                             preferred_element_type=f32) + b0_ref[...][None, :]
    g = jax.lax.dot_general(h0, wg_ref[...], (((1,), (1,)), ((), ())),
                            preferred_element_type=f32) + bg_ref[...][None, :]
    u = jax.lax.dot_general(h0, wu_ref[...], (((1,), (1,)), ((), ())),
                            preferred_element_type=f32) + bu_ref[...][None, :]
    h = (g * jax.nn.sigmoid(g)) * u
    h_ref[...] = h
    logits = jax.lax.dot_general(h, wgate_ref[...], (((1,), (1,)), ((), ())),
                                 preferred_element_type=f32)
    m = jnp.max(logits, axis=-1, keepdims=True)
    p = jnp.exp(logits - m)
    p = p / jnp.sum(p, axis=-1, keepdims=True)
    cols = jax.lax.broadcasted_iota(jnp.int32, p.shape, 1)
    tws, tis = [], []
    for _ in range(K):
        mx = jnp.max(p, axis=-1, keepdims=True)
        idx = jnp.min(jnp.where(p == mx, cols, E), axis=-1, keepdims=True)
        tws.append(mx)
        tis.append(idx)
        p = jnp.where(cols == idx, -1.0, p)
    tw_ref[...] = jnp.concatenate(tws, axis=-1)
    ti_ref[...] = jnp.concatenate(tis, axis=-1)


def _route_kernel(ti_ref, pos_ref, post_ref, te_ref):
    f32, i32, bf16 = jnp.float32, jnp.int32, jnp.bfloat16
    ti = ti_ref[...]
    iota_e = jax.lax.broadcasted_iota(i32, (TOKENS, E), 1)
    ohs = [(ti[:, k:k + 1] == iota_e).astype(f32) for k in range(K)]
    rowsum = ohs[0] + ohs[1] + ohs[2] + ohs[3]
    ri = jax.lax.broadcasted_iota(i32, (TOKENS, TOKENS), 0)
    ci = jax.lax.broadcasted_iota(i32, (TOKENS, TOKENS), 1)
    tril = (ci < ri).astype(bf16)
    prefix = jnp.dot(tril, rowsum.astype(bf16), preferred_element_type=f32)
    counts = jnp.sum(rowsum, axis=0, keepdims=True)
    c_pad = ((counts.astype(i32) + BT - 1) // BT) * BT
    ue = jax.lax.broadcasted_iota(i32, (E, E), 0)
    ve = jax.lax.broadcasted_iota(i32, (E, E), 1)
    upper = (ue < ve).astype(f32)
    pstart = jnp.dot(c_pad.astype(f32), upper,
                     preferred_element_type=f32)
    poss = []
    for k in range(K):
        r_k = jnp.sum(ohs[k] * prefix, axis=1, keepdims=True)
        p_k = jnp.sum(ohs[k] * pstart, axis=1, keepdims=True)
        poss.append(r_k + p_k)
    pos_f = jnp.concatenate(poss, axis=1)
    pos_ref[...] = pos_f.astype(i32)
    post_ref[...] = jnp.transpose(pos_f).astype(i32)
    base = (jax.lax.broadcasted_iota(i32, (NT, 1), 0) * BT).astype(f32)
    cmp = (pstart <= base).astype(i32)
    te_ref[...] = (jnp.sum(cmp, axis=1, keepdims=True) - 1).astype(i32)


def _expert_kernel(te_ref, pos_ref, h_ref, w1_ref, w3_ref, w2_ref, rows_ref):
    del te_ref
    f32, i32, bf16 = jnp.float32, jnp.int32, jnp.bfloat16
    i = pl.program_id(0)
    sid = jax.lax.broadcasted_iota(i32, (BT, TOKENS), 0) + i * BT
    dbool = (pos_ref[0:1, :] == sid)
    for k in range(1, K):
        dbool = dbool | (pos_ref[k:k + 1, :] == sid)
    onehot = dbool.astype(bf16)
    hs = jnp.dot(onehot, h_ref[...], preferred_element_type=f32)
    g = jax.lax.dot_general(hs, w1_ref[0], (((1,), (1,)), ((), ())),
                            preferred_element_type=f32)
    u = jax.lax.dot_general(hs, w3_ref[0], (((1,), (1,)), ((), ())),
                            preferred_element_type=f32)
    a = ((g * jax.nn.sigmoid(g)) * u).astype(bf16)
    rows_ref[...] = jax.lax.dot_general(
        a, w2_ref[0], (((1,), (1,)), ((), ())),
        preferred_element_type=f32).astype(bf16)


def _combine_kernel(pos_ref, tw_ref, rows_ref, y_ref):
    j = pl.program_id(1)

    @pl.when(j == 0)
    def _():
        y_ref[...] = jnp.zeros_like(y_ref)

    base = j * PC
    iota = jax.lax.broadcasted_iota(jnp.int32, (TT, PC), 1) + base
    m = jnp.zeros((TT, PC), jnp.float32)
    for k in range(K):
        pk = pos_ref[:, k][:, None]
        wk = tw_ref[:, k][:, None]
        m = m + jnp.where(pk == iota, wk, 0.0)
    y_ref[...] += jnp.dot(m.astype(jnp.bfloat16), rows_ref[...],
                          preferred_element_type=jnp.float32)


def _shared_kernel(h_ref, w1_ref, w3_ref, w2_ref, y_ref, out_ref):
    f32 = jnp.float32
    h = h_ref[...]
    g = jax.lax.dot_general(h, w1_ref[...], (((1,), (1,)), ((), ())),
                            preferred_element_type=f32)
    u = jax.lax.dot_general(h, w3_ref[...], (((1,), (1,)), ((), ())),
                            preferred_element_type=f32)
    a = ((g * jax.nn.sigmoid(g)) * u).astype(jnp.bfloat16)
    out_ref[...] = y_ref[...] + jax.lax.dot_general(
        a, w2_ref[...], (((1,), (1,)), ((), ())),
        preferred_element_type=f32)


def kernel(x, W0, b0, Wg, bg, Wu, bu, Wgate, we1, we3, we2, ws1, ws3, ws2):
    f32, bf16, i32 = jnp.float32, jnp.bfloat16, jnp.int32

    h, tw, ti = pl.pallas_call(
        _front_kernel,
        grid=(NTT,),
        in_specs=[
            pl.BlockSpec((TT, F), lambda i: (i, 0)),
            pl.BlockSpec((F, F), lambda i: (0, 0)),
            pl.BlockSpec((F,), lambda i: (0,)),
            pl.BlockSpec((F, F), lambda i: (0, 0)),
            pl.BlockSpec((F,), lambda i: (0,)),
            pl.BlockSpec((F, F), lambda i: (0, 0)),
            pl.BlockSpec((F,), lambda i: (0,)),
            pl.BlockSpec((E, F), lambda i: (0, 0)),
        ],
        out_specs=[
            pl.BlockSpec((TT, F), lambda i: (i, 0)),
            pl.BlockSpec((TT, K), lambda i: (i, 0)),
            pl.BlockSpec((TT, K), lambda i: (i, 0)),
        ],
        out_shape=[
            jax.ShapeDtypeStruct((TOKENS, F), f32),
            jax.ShapeDtypeStruct((TOKENS, K), f32),
            jax.ShapeDtypeStruct((TOKENS, K), i32),
        ],
    )(x, W0, b0, Wg, bg, Wu, bu, Wgate)

    pos, pos_t, te = pl.pallas_call(
        _route_kernel,
        grid=(1,),
        in_specs=[pl.BlockSpec((TOKENS, K), lambda i: (0, 0))],
        out_specs=[
            pl.BlockSpec((TOKENS, K), lambda i: (0, 0)),
            pl.BlockSpec((K, TOKENS), lambda i: (0, 0)),
            pl.BlockSpec((NT, 1), lambda i: (0, 0)),
        ],
        out_shape=[
            jax.ShapeDtypeStruct((TOKENS, K), i32),
            jax.ShapeDtypeStruct((K, TOKENS), i32),
            jax.ShapeDtypeStruct((NT, 1), i32),
        ],
    )(ti)

    h_b = h.astype(bf16)
    we2_b = we2.astype(bf16)
    ws1_b, ws3_b, ws2_b = ws1.astype(bf16), ws3.astype(bf16), ws2.astype(bf16)

    rows = pl.pallas_call(
        _expert_kernel,
        grid_spec=pltpu.PrefetchScalarGridSpec(
            num_scalar_prefetch=1,
            grid=(NT,),
            in_specs=[
                pl.BlockSpec((K, TOKENS), lambda i, te: (0, 0)),
                pl.BlockSpec((TOKENS, F), lambda i, te: (0, 0)),
                pl.BlockSpec((1, H, F), lambda i, te: (te[i, 0], 0, 0)),
                pl.BlockSpec((1, H, F), lambda i, te: (te[i, 0], 0, 0)),
                pl.BlockSpec((1, F, H), lambda i, te: (te[i, 0], 0, 0)),
            ],
            out_specs=pl.BlockSpec((BT, F), lambda i, te: (i, 0)),
        ),
        out_shape=jax.ShapeDtypeStruct((P, F), bf16),
    )(te, pos_t, h_b, we1, we3, we2_b)

    y = pl.pallas_call(
        _combine_kernel,
        grid=(NTT, NPC),
        in_specs=[
            pl.BlockSpec((TT, K), lambda i, j: (i, 0)),
            pl.BlockSpec((TT, K), lambda i, j: (i, 0)),
            pl.BlockSpec((PC, F), lambda i, j: (j, 0)),
        ],
        out_specs=pl.BlockSpec((TT, F), lambda i, j: (i, 0)),
        out_shape=jax.ShapeDtypeStruct((TOKENS, F), f32),
    )(pos, tw, rows)

    out = pl.pallas_call(
        _shared_kernel,
        grid=(NTT,),
        in_specs=[
            pl.BlockSpec((TT, F), lambda i: (i, 0)),
            pl.BlockSpec((SH, F), lambda i: (0, 0)),
            pl.BlockSpec((SH, F), lambda i: (0, 0)),
            pl.BlockSpec((F, SH), lambda i: (0, 0)),
            pl.BlockSpec((TT, F), lambda i: (i, 0)),
        ],
        out_specs=pl.BlockSpec((TT, F), lambda i: (i, 0)),
        out_shape=jax.ShapeDtypeStruct((TOKENS, F), f32),
    )(h_b, ws1_b, ws3_b, ws2_b, y)

    return out


# speedup vs baseline: 1.1509x; 1.0184x over previous
import functools

import jax
import jax.numpy as jnp
from jax.experimental import pallas as pl
from jax.experimental.pallas import tpu as pltpu

TOKENS = 2048
F = 768
E = 16
K = 4
H = 3072
SH = 6144
BT = 128
P = 10240
NT = P // BT
TT = 256
NTT = TOKENS // TT
PC = 2048
NPC = P // PC


def _front_kernel(x_ref, w0_ref, b0_ref, wg_ref, bg_ref, wu_ref, bu_ref,
                  wgate_ref, h_ref, tw_ref, ti_ref):
    f32 = jnp.float32
    x = x_ref[...]
    h0 = jax.lax.dot_general(x, w0_ref[...], (((1,), (1,)), ((), ())),
                             preferred_element_type=f32) + b0_ref[...][None, :]
    g = jax.lax.dot_general(h0, wg_ref[...], (((1,), (1,)), ((), ())),
                            preferred_element_type=f32) + bg_ref[...][None, :]
    u = jax.lax.dot_general(h0, wu_ref[...], (((1,), (1,)), ((), ())),
                            preferred_element_type=f32) + bu_ref[...][None, :]
    h = (g * jax.nn.sigmoid(g)) * u
    h_ref[...] = h
    logits = jax.lax.dot_general(h, wgate_ref[...], (((1,), (1,)), ((), ())),
                                 preferred_element_type=f32)
    m = jnp.max(logits, axis=-1, keepdims=True)
    p = jnp.exp(logits - m)
    p = p / jnp.sum(p, axis=-1, keepdims=True)
    cols = jax.lax.broadcasted_iota(jnp.int32, p.shape, 1)
    tws, tis = [], []
    for _ in range(K):
        mx = jnp.max(p, axis=-1, keepdims=True)
        idx = jnp.min(jnp.where(p == mx, cols, E), axis=-1, keepdims=True)
        tws.append(mx)
        tis.append(idx)
        p = jnp.where(cols == idx, -1.0, p)
    tw_ref[...] = jnp.concatenate(tws, axis=-1)
    ti_ref[...] = jnp.concatenate(tis, axis=-1)


def _route_kernel(ti_ref, pos_ref, post_ref, te_ref):
    f32, i32, bf16 = jnp.float32, jnp.int32, jnp.bfloat16
    ti = ti_ref[...]
    iota_e = jax.lax.broadcasted_iota(i32, (TOKENS, E), 1)
    ohs = [(ti[:, k:k + 1] == iota_e).astype(f32) for k in range(K)]
    rowsum = ohs[0] + ohs[1] + ohs[2] + ohs[3]
    ri = jax.lax.broadcasted_iota(i32, (TOKENS, TOKENS), 0)
    ci = jax.lax.broadcasted_iota(i32, (TOKENS, TOKENS), 1)
    tril = (ci < ri).astype(bf16)
    prefix = jnp.dot(tril, rowsum.astype(bf16), preferred_element_type=f32)
    counts = jnp.sum(rowsum, axis=0, keepdims=True)
    c_pad = ((counts.astype(i32) + BT - 1) // BT) * BT
    ue = jax.lax.broadcasted_iota(i32, (E, E), 0)
    ve = jax.lax.broadcasted_iota(i32, (E, E), 1)
    upper = (ue < ve).astype(f32)
    pstart = jnp.dot(c_pad.astype(f32), upper,
                     preferred_element_type=f32)
    poss = []
    for k in range(K):
        r_k = jnp.sum(ohs[k] * prefix, axis=1, keepdims=True)
        p_k = jnp.sum(ohs[k] * pstart, axis=1, keepdims=True)
        poss.append(r_k + p_k)
    pos_f = jnp.concatenate(poss, axis=1)
    pos_ref[...] = pos_f.astype(i32)
    post_ref[...] = jnp.transpose(pos_f).astype(i32)
    base = (jax.lax.broadcasted_iota(i32, (NT, 1), 0) * BT).astype(f32)
    cmp = (pstart <= base).astype(i32)
    te_ref[...] = (jnp.sum(cmp, axis=1, keepdims=True) - 1).astype(i32)


def _expert_kernel(te_ref, pos_ref, h_ref, w1_ref, w3_ref, w2_ref, rows_ref):
    del te_ref
    f32, i32, bf16 = jnp.float32, jnp.int32, jnp.bfloat16
    i = pl.program_id(0)
    sid = jax.lax.broadcasted_iota(i32, (BT, TOKENS), 0) + i * BT
    dbool = (pos_ref[0:1, :] == sid)
    for k in range(1, K):
        dbool = dbool | (pos_ref[k:k + 1, :] == sid)
    onehot = dbool.astype(bf16)
    hs = jnp.dot(onehot, h_ref[...], preferred_element_type=f32)
    g = jax.lax.dot_general(hs, w1_ref[0], (((1,), (1,)), ((), ())),
                            preferred_element_type=f32)
    u = jax.lax.dot_general(hs, w3_ref[0], (((1,), (1,)), ((), ())),
                            preferred_element_type=f32)
    a = ((g * jax.nn.sigmoid(g)) * u).astype(bf16)
    rows_ref[...] = jax.lax.dot_general(
        a, w2_ref[0], (((1,), (1,)), ((), ())),
        preferred_element_type=f32).astype(bf16)


def _combine_kernel(pos_ref, tw_ref, rows_ref, y_ref):
    iota = jax.lax.broadcasted_iota(jnp.int32, (TT, P), 1)
    m = jnp.zeros((TT, P), jnp.float32)
    for k in range(K):
        pk = pos_ref[:, k][:, None]
        wk = tw_ref[:, k][:, None]
        m = m + jnp.where(pk == iota, wk, 0.0)
    y_ref[...] = jnp.dot(m.astype(jnp.bfloat16), rows_ref[...],
                         preferred_element_type=jnp.float32)


def _shared_kernel(h_ref, w1_ref, w3_ref, w2_ref, y_ref, out_ref):
    f32 = jnp.float32
    h = h_ref[...]
    g = jax.lax.dot_general(h, w1_ref[...], (((1,), (1,)), ((), ())),
                            preferred_element_type=f32)
    u = jax.lax.dot_general(h, w3_ref[...], (((1,), (1,)), ((), ())),
                            preferred_element_type=f32)
    a = ((g * jax.nn.sigmoid(g)) * u).astype(jnp.bfloat16)
    out_ref[...] = y_ref[...] + jax.lax.dot_general(
        a, w2_ref[...], (((1,), (1,)), ((), ())),
        preferred_element_type=f32)


def kernel(x, W0, b0, Wg, bg, Wu, bu, Wgate, we1, we3, we2, ws1, ws3, ws2):
    f32, bf16, i32 = jnp.float32, jnp.bfloat16, jnp.int32

    h, tw, ti = pl.pallas_call(
        _front_kernel,
        grid=(NTT,),
        in_specs=[
            pl.BlockSpec((TT, F), lambda i: (i, 0)),
            pl.BlockSpec((F, F), lambda i: (0, 0)),
            pl.BlockSpec((F,), lambda i: (0,)),
            pl.BlockSpec((F, F), lambda i: (0, 0)),
            pl.BlockSpec((F,), lambda i: (0,)),
            pl.BlockSpec((F, F), lambda i: (0, 0)),
            pl.BlockSpec((F,), lambda i: (0,)),
            pl.BlockSpec((E, F), lambda i: (0, 0)),
        ],
        out_specs=[
            pl.BlockSpec((TT, F), lambda i: (i, 0)),
            pl.BlockSpec((TT, K), lambda i: (i, 0)),
            pl.BlockSpec((TT, K), lambda i: (i, 0)),
        ],
        out_shape=[
            jax.ShapeDtypeStruct((TOKENS, F), f32),
            jax.ShapeDtypeStruct((TOKENS, K), f32),
            jax.ShapeDtypeStruct((TOKENS, K), i32),
        ],
    )(x, W0, b0, Wg, bg, Wu, bu, Wgate)

    pos, pos_t, te = pl.pallas_call(
        _route_kernel,
        grid=(1,),
        in_specs=[pl.BlockSpec((TOKENS, K), lambda i: (0, 0))],
        out_specs=[
            pl.BlockSpec((TOKENS, K), lambda i: (0, 0)),
            pl.BlockSpec((K, TOKENS), lambda i: (0, 0)),
            pl.BlockSpec((NT, 1), lambda i: (0, 0)),
        ],
        out_shape=[
            jax.ShapeDtypeStruct((TOKENS, K), i32),
            jax.ShapeDtypeStruct((K, TOKENS), i32),
            jax.ShapeDtypeStruct((NT, 1), i32),
        ],
    )(ti)

    h_b = h.astype(bf16)
    we2_b = we2.astype(bf16)
    ws1_b, ws3_b, ws2_b = ws1.astype(bf16), ws3.astype(bf16), ws2.astype(bf16)

    rows = pl.pallas_call(
        _expert_kernel,
        grid_spec=pltpu.PrefetchScalarGridSpec(
            num_scalar_prefetch=1,
            grid=(NT,),
            in_specs=[
                pl.BlockSpec((K, TOKENS), lambda i, te: (0, 0)),
                pl.BlockSpec((TOKENS, F), lambda i, te: (0, 0)),
                pl.BlockSpec((1, H, F), lambda i, te: (te[i, 0], 0, 0)),
                pl.BlockSpec((1, H, F), lambda i, te: (te[i, 0], 0, 0)),
                pl.BlockSpec((1, F, H), lambda i, te: (te[i, 0], 0, 0)),
            ],
            out_specs=pl.BlockSpec((BT, F), lambda i, te: (i, 0)),
        ),
        out_shape=jax.ShapeDtypeStruct((P, F), bf16),
    )(te, pos_t, h_b, we1, we3, we2_b)

    y = pl.pallas_call(
        _combine_kernel,
        grid=(NTT,),
        in_specs=[
            pl.BlockSpec((TT, K), lambda i: (i, 0)),
            pl.BlockSpec((TT, K), lambda i: (i, 0)),
            pl.BlockSpec((P, F), lambda i: (0, 0)),
        ],
        out_specs=pl.BlockSpec((TT, F), lambda i: (i, 0)),
        out_shape=jax.ShapeDtypeStruct((TOKENS, F), f32),
    )(pos, tw, rows)

    out = pl.pallas_call(
        _shared_kernel,
        grid=(NTT,),
        in_specs=[
            pl.BlockSpec((TT, F), lambda i: (i, 0)),
            pl.BlockSpec((SH, F), lambda i: (0, 0)),
            pl.BlockSpec((SH, F), lambda i: (0, 0)),
            pl.BlockSpec((F, SH), lambda i: (0, 0)),
            pl.BlockSpec((TT, F), lambda i: (i, 0)),
        ],
        out_specs=pl.BlockSpec((TT, F), lambda i: (i, 0)),
        out_shape=jax.ShapeDtypeStruct((TOKENS, F), f32),
    )(h_b, ws1_b, ws3_b, ws2_b, y)

    return out


# speedup vs baseline: 1.4305x; 1.2429x over previous
import functools

import jax
import jax.numpy as jnp
from jax.experimental import pallas as pl
from jax.experimental.pallas import tpu as pltpu

TOKENS = 2048
F = 768
E = 16
K = 4
H = 3072
SH = 6144
BT = 256
P = 12288
NT = P // BT
TT = 256
NTT = TOKENS // TT
PC = 2048
NPC = P // PC


def _front_kernel(x_ref, w0_ref, b0_ref, wg_ref, bg_ref, wu_ref, bu_ref,
                  wgate_ref, h_ref, tw_ref, ti_ref):
    f32 = jnp.float32
    x = x_ref[...]
    h0 = jax.lax.dot_general(x, w0_ref[...], (((1,), (1,)), ((), ())),
                             preferred_element_type=f32) + b0_ref[...][None, :]
    g = jax.lax.dot_general(h0, wg_ref[...], (((1,), (1,)), ((), ())),
                            preferred_element_type=f32) + bg_ref[...][None, :]
    u = jax.lax.dot_general(h0, wu_ref[...], (((1,), (1,)), ((), ())),
                            preferred_element_type=f32) + bu_ref[...][None, :]
    h = (g * jax.nn.sigmoid(g)) * u
    h_ref[...] = h
    logits = jax.lax.dot_general(h, wgate_ref[...], (((1,), (1,)), ((), ())),
                                 preferred_element_type=f32)
    m = jnp.max(logits, axis=-1, keepdims=True)
    p = jnp.exp(logits - m)
    p = p / jnp.sum(p, axis=-1, keepdims=True)
    cols = jax.lax.broadcasted_iota(jnp.int32, p.shape, 1)
    tws, tis = [], []
    for _ in range(K):
        mx = jnp.max(p, axis=-1, keepdims=True)
        idx = jnp.min(jnp.where(p == mx, cols, E), axis=-1, keepdims=True)
        tws.append(mx)
        tis.append(idx)
        p = jnp.where(cols == idx, -1.0, p)
    tw_ref[...] = jnp.concatenate(tws, axis=-1)
    ti_ref[...] = jnp.concatenate(tis, axis=-1)


def _route_kernel(ti_ref, pos_ref, post_ref, te_ref):
    f32, i32, bf16 = jnp.float32, jnp.int32, jnp.bfloat16
    ti = ti_ref[...]
    iota_e = jax.lax.broadcasted_iota(i32, (TOKENS, E), 1)
    ohs = [(ti[:, k:k + 1] == iota_e).astype(f32) for k in range(K)]
    rowsum = ohs[0] + ohs[1] + ohs[2] + ohs[3]
    ri = jax.lax.broadcasted_iota(i32, (TOKENS, TOKENS), 0)
    ci = jax.lax.broadcasted_iota(i32, (TOKENS, TOKENS), 1)
    tril = (ci < ri).astype(bf16)
    prefix = jnp.dot(tril, rowsum.astype(bf16), preferred_element_type=f32)
    counts = jnp.sum(rowsum, axis=0, keepdims=True)
    c_pad = ((counts.astype(i32) + BT - 1) // BT) * BT
    ue = jax.lax.broadcasted_iota(i32, (E, E), 0)
    ve = jax.lax.broadcasted_iota(i32, (E, E), 1)
    upper = (ue < ve).astype(f32)
    pstart = jnp.dot(c_pad.astype(f32), upper,
                     preferred_element_type=f32)
    poss = []
    for k in range(K):
        r_k = jnp.sum(ohs[k] * prefix, axis=1, keepdims=True)
        p_k = jnp.sum(ohs[k] * pstart, axis=1, keepdims=True)
        poss.append(r_k + p_k)
    pos_f = jnp.concatenate(poss, axis=1)
    pos_ref[...] = pos_f.astype(i32)
    post_ref[...] = jnp.transpose(pos_f).astype(i32)
    base = (jax.lax.broadcasted_iota(i32, (NT, 1), 0) * BT).astype(f32)
    cmp = (pstart <= base).astype(i32)
    te_ref[...] = (jnp.sum(cmp, axis=1, keepdims=True) - 1).astype(i32)


def _expert_kernel(te_ref, pos_ref, h_ref, w1_ref, w3_ref, w2_ref, rows_ref):
    del te_ref
    f32, i32, bf16 = jnp.float32, jnp.int32, jnp.bfloat16
    i = pl.program_id(0)
    sid = jax.lax.broadcasted_iota(i32, (BT, TOKENS), 0) + i * BT
    dbool = (pos_ref[0:1, :] == sid)
    for k in range(1, K):
        dbool = dbool | (pos_ref[k:k + 1, :] == sid)
    onehot = dbool.astype(bf16)
    hs = jnp.dot(onehot, h_ref[...], preferred_element_type=f32)
    g = jax.lax.dot_general(hs, w1_ref[0], (((1,), (1,)), ((), ())),
                            preferred_element_type=f32)
    u = jax.lax.dot_general(hs, w3_ref[0], (((1,), (1,)), ((), ())),
                            preferred_element_type=f32)
    a = ((g * jax.nn.sigmoid(g)) * u).astype(bf16)
    rows_ref[...] = jax.lax.dot_general(
        a, w2_ref[0], (((1,), (1,)), ((), ())),
        preferred_element_type=f32).astype(bf16)


def _combine_kernel(pos_ref, tw_ref, rows_ref, y_ref):
    iota = jax.lax.broadcasted_iota(jnp.int32, (TT, P), 1)
    m = jnp.zeros((TT, P), jnp.float32)
    for k in range(K):
        pk = pos_ref[:, k][:, None]
        wk = tw_ref[:, k][:, None]
        m = m + jnp.where(pk == iota, wk, 0.0)
    y_ref[...] = jnp.dot(m.astype(jnp.bfloat16), rows_ref[...],
                         preferred_element_type=jnp.float32)


def _shared_kernel(h_ref, w1_ref, w3_ref, w2_ref, y_ref, out_ref):
    f32 = jnp.float32
    h = h_ref[...]
    g = jax.lax.dot_general(h, w1_ref[...], (((1,), (1,)), ((), ())),
                            preferred_element_type=f32)
    u = jax.lax.dot_general(h, w3_ref[...], (((1,), (1,)), ((), ())),
                            preferred_element_type=f32)
    a = ((g * jax.nn.sigmoid(g)) * u).astype(jnp.bfloat16)
    out_ref[...] = y_ref[...] + jax.lax.dot_general(
        a, w2_ref[...], (((1,), (1,)), ((), ())),
        preferred_element_type=f32)


def kernel(x, W0, b0, Wg, bg, Wu, bu, Wgate, we1, we3, we2, ws1, ws3, ws2):
    f32, bf16, i32 = jnp.float32, jnp.bfloat16, jnp.int32

    h, tw, ti = pl.pallas_call(
        _front_kernel,
        grid=(NTT,),
        in_specs=[
            pl.BlockSpec((TT, F), lambda i: (i, 0)),
            pl.BlockSpec((F, F), lambda i: (0, 0)),
            pl.BlockSpec((F,), lambda i: (0,)),
            pl.BlockSpec((F, F), lambda i: (0, 0)),
            pl.BlockSpec((F,), lambda i: (0,)),
            pl.BlockSpec((F, F), lambda i: (0, 0)),
            pl.BlockSpec((F,), lambda i: (0,)),
            pl.BlockSpec((E, F), lambda i: (0, 0)),
        ],
        out_specs=[
            pl.BlockSpec((TT, F), lambda i: (i, 0)),
            pl.BlockSpec((TT, K), lambda i: (i, 0)),
            pl.BlockSpec((TT, K), lambda i: (i, 0)),
        ],
        out_shape=[
            jax.ShapeDtypeStruct((TOKENS, F), f32),
            jax.ShapeDtypeStruct((TOKENS, K), f32),
            jax.ShapeDtypeStruct((TOKENS, K), i32),
        ],
    )(x, W0, b0, Wg, bg, Wu, bu, Wgate)

    pos, pos_t, te = pl.pallas_call(
        _route_kernel,
        grid=(1,),
        in_specs=[pl.BlockSpec((TOKENS, K), lambda i: (0, 0))],
        out_specs=[
            pl.BlockSpec((TOKENS, K), lambda i: (0, 0)),
            pl.BlockSpec((K, TOKENS), lambda i: (0, 0)),
            pl.BlockSpec((NT, 1), lambda i: (0, 0)),
        ],
        out_shape=[
            jax.ShapeDtypeStruct((TOKENS, K), i32),
            jax.ShapeDtypeStruct((K, TOKENS), i32),
            jax.ShapeDtypeStruct((NT, 1), i32),
        ],
    )(ti)

    h_b = h.astype(bf16)
    we2_b = we2.astype(bf16)
    ws1_b, ws3_b, ws2_b = ws1.astype(bf16), ws3.astype(bf16), ws2.astype(bf16)

    rows = pl.pallas_call(
        _expert_kernel,
        grid_spec=pltpu.PrefetchScalarGridSpec(
            num_scalar_prefetch=1,
            grid=(NT,),
            in_specs=[
                pl.BlockSpec((K, TOKENS), lambda i, te: (0, 0)),
                pl.BlockSpec((TOKENS, F), lambda i, te: (0, 0)),
                pl.BlockSpec((1, H, F), lambda i, te: (te[i, 0], 0, 0)),
                pl.BlockSpec((1, H, F), lambda i, te: (te[i, 0], 0, 0)),
                pl.BlockSpec((1, F, H), lambda i, te: (te[i, 0], 0, 0)),
            ],
            out_specs=pl.BlockSpec((BT, F), lambda i, te: (i, 0)),
        ),
        out_shape=jax.ShapeDtypeStruct((P, F), bf16),
    )(te, pos_t, h_b, we1, we3, we2_b)

    y = pl.pallas_call(
        _combine_kernel,
        grid=(NTT,),
        in_specs=[
            pl.BlockSpec((TT, K), lambda i: (i, 0)),
            pl.BlockSpec((TT, K), lambda i: (i, 0)),
            pl.BlockSpec((P, F), lambda i: (0, 0)),
        ],
        out_specs=pl.BlockSpec((TT, F), lambda i: (i, 0)),
        out_shape=jax.ShapeDtypeStruct((TOKENS, F), f32),
    )(pos, tw, rows)

    out = pl.pallas_call(
        _shared_kernel,
        grid=(NTT,),
        in_specs=[
            pl.BlockSpec((TT, F), lambda i: (i, 0)),
            pl.BlockSpec((SH, F), lambda i: (0, 0)),
            pl.BlockSpec((SH, F), lambda i: (0, 0)),
            pl.BlockSpec((F, SH), lambda i: (0, 0)),
            pl.BlockSpec((TT, F), lambda i: (i, 0)),
        ],
        out_specs=pl.BlockSpec((TT, F), lambda i: (i, 0)),
        out_shape=jax.ShapeDtypeStruct((TOKENS, F), f32),
    )(h_b, ws1_b, ws3_b, ws2_b, y)

    return out
